# baseline (speedup 1.0000x reference)
import functools

import jax
import jax.numpy as jnp
from jax import lax
from jax.experimental import pallas as pl
from jax.experimental.pallas import tpu as pltpu
from jax.experimental.pallas import tpu_sc as plsc

_NC = 2
_NS = 16
_L = 16
_CHUNK = 128
_ZR = 256


def _row_partition(N):
    per = (N // _NS) & ~7
    extra = N - _NS * per
    return per, extra


def _zero_accumulator(zbuf, acc, sid, N):
    per, extra = _row_partition(N)
    def initz(i, c):
        zbuf[i, :] = jnp.zeros((_L,), jnp.float32)
        return c
    lax.fori_loop(0, _ZR, initz, 0)
    base = sid * per
    span = per + extra
    nz, zt = divmod(span, _ZR)
    def zc(j, c):
        pltpu.sync_copy(zbuf, acc.at[pl.ds(base + j * _ZR, _ZR)])
        return c
    lax.fori_loop(0, nz, zc, 0)
    if zt:
        pltpu.sync_copy(zbuf.at[pl.ds(0, zt)], acc.at[pl.ds(base + nz * _ZR, zt)])


def _writeout(acc, o0, o1, cid, sid, N):
    per, extra = _row_partition(N)
    base = sid * per
    sl = pl.ds(base, per)
    tl = pl.ds(_NS * per, max(extra, 8))

    @pl.when(cid == 0)
    def _():
        pltpu.sync_copy(acc.at[sl], o0.at[sl])

    @pl.when(cid == 1)
    def _():
        pltpu.sync_copy(acc.at[sl], o1.at[sl])

    if extra:
        @pl.when((cid == 0) & (sid == _NS - 1))
        def _():
            pltpu.sync_copy(acc.at[tl], o0.at[tl])

        @pl.when((cid == 1) & (sid == _NS - 1))
        def _():
            pltpu.sync_copy(acc.at[tl], o1.at[tl])


_K = 4


def _pad_chunks(E):
    q = _NC * _NS * _K
    return -(-(E // _CHUNK + (1 if E % _CHUNK else 0)) // q) * q


@functools.lru_cache(maxsize=None)
def _count_kernel(N, E):
    C = _pad_chunks(E)
    G = C // (_NC * _NS * _K)
    mesh = plsc.VectorSubcoreMesh(core_axis_name="c", subcore_axis_name="s")

    @functools.partial(
        pl.kernel, mesh=mesh,
        out_type=(jax.ShapeDtypeStruct((N, _L), jnp.float32),
                  jax.ShapeDtypeStruct((N, _L), jnp.float32)),
        scratch_types=[
            pltpu.VMEM((_K, _CHUNK), jnp.int32),
            pltpu.VMEM((_K, _CHUNK), jnp.int32),
            pltpu.VMEM((_CHUNK, _L), jnp.float32),
            pltpu.VMEM((_ZR, _L), jnp.float32),
            pltpu.VMEM_SHARED((N + 8, _L), jnp.float32),
            pltpu.SemaphoreType.DMA,
            pltpu.SemaphoreType.DMA,
        ],
        compiler_params=pltpu.CompilerParams(use_tc_tiling_on_sc=False),
    )
    def k(dst2d_hbm, c0_hbm, c1_hbm, didx0, didx1, ones, zbuf, acc, ss0, ss1):
        cid = lax.axis_index("c")
        sid = lax.axis_index("s")

        def inito(i, c):
            ones[i, :] = jnp.full((_L,), 1.0, jnp.float32)
            return c
        lax.fori_loop(0, _CHUNK, inito, 0)
        _zero_accumulator(zbuf, acc, sid, N)
        plsc.subcore_barrier()

        row0 = cid * (C // _NC) + sid * (G * _K)
        dummy = c0_hbm.at[pl.ds(0, _CHUNK)]

        def phase(i, didx, ss):
            @pl.when((i >= 2) & (i <= G + 1))
            def _():
                for b in range(_K):
                    pltpu.make_async_copy(dummy, ones, ss).wait()

            @pl.when(i < G)
            def _():
                pltpu.sync_copy(dst2d_hbm.at[pl.ds(row0 + i * _K, _K)], didx)
                for b in range(_K):
                    pltpu.async_copy(ones, acc.at[didx.at[b]], ss, add=True)

        def body(t, c):
            phase(2 * t, didx0, ss0)
            phase(2 * t + 1, didx1, ss1)
            return c
        lax.fori_loop(0, (G + 3) // 2, body, 0)

        plsc.subcore_barrier()
        _writeout(acc, c0_hbm, c1_hbm, cid, sid, N)

    return k


@functools.lru_cache(maxsize=None)
def _gather_scatter_kernel(N, E):
    C = _pad_chunks(E)
    G = C // (_NS * _K)
    mesh = plsc.VectorSubcoreMesh(core_axis_name="c", subcore_axis_name="s")

    @functools.partial(
        pl.kernel, mesh=mesh,
        out_type=(jax.ShapeDtypeStruct((N, _L), jnp.float32),
                  jax.ShapeDtypeStruct((N, _L), jnp.float32)),
        scratch_types=[
            pltpu.VMEM((_K * _CHUNK,), jnp.int32),
            pltpu.VMEM((_K * _CHUNK,), jnp.int32),
            pltpu.VMEM((_K, _CHUNK), jnp.int32),
            pltpu.VMEM((_K, _CHUNK), jnp.int32),
            pltpu.VMEM((_K, _CHUNK, _L), jnp.float32),
            pltpu.VMEM((_K, _CHUNK, _L), jnp.float32),
            pltpu.VMEM((_ZR, _L), jnp.float32),
            pltpu.VMEM_SHARED((N + 8, _L), jnp.float32),
            pltpu.SemaphoreType.DMA,
            pltpu.SemaphoreType.DMA,
            pltpu.SemaphoreType.DMA,
            pltpu.SemaphoreType.DMA,
            pltpu.SemaphoreType.DMA,
            pltpu.SemaphoreType.DMA,
        ],
        compiler_params=pltpu.CompilerParams(use_tc_tiling_on_sc=False),
    )
    def k(src_hbm, dst2d_hbm, tab_hbm, o0_hbm, o1_hbm,
          sidx0, sidx1, didx0, didx1, rows0, rows1, zbuf, acc,
          sg0, sg1, ss0, ss1, si0, si1):
        cid = lax.axis_index("c")
        sid = lax.axis_index("s")

        _zero_accumulator(zbuf, acc, sid, N)
        plsc.subcore_barrier()

        chunk0 = sid * (G * _K)
        off = cid
        dummy = o0_hbm.at[pl.ds(0, _CHUNK)]
        dummy_s = src_hbm.at[pl.ds(0, _K * _CHUNK)]
        dummy_d = dst2d_hbm.at[pl.ds(0, _K)]

        def fire_idx(i, sidx, didx, si):
            e0 = pl.multiple_of(chunk0 * _CHUNK + i * (_K * _CHUNK), 8)
            pltpu.async_copy(src_hbm.at[pl.ds(e0, _K * _CHUNK)], sidx, si)
            pltpu.async_copy(dst2d_hbm.at[pl.ds(chunk0 + i * _K, _K)],
                             didx, si)

        def gather(sidx, rows, si, sg):
            pltpu.make_async_copy(dummy_s, sidx, si).wait()
            pltpu.make_async_copy(dummy_d, didx0, si).wait()

            def f(j, c):
                sl = pl.ds(j * _L, _L)
                sidx[sl] = sidx[sl] * 2 + off
                return c
            lax.fori_loop(0, _K * _CHUNK // _L, f, 0)
            for b in range(_K):
                pltpu.async_copy(
                    tab_hbm.at[sidx.at[pl.ds(b * _CHUNK, _CHUNK)]],
                    rows.at[b], sg)

        def drain_and_scatter(didx, rows, sg, ss):
            for b in range(_K):
                pltpu.make_async_copy(dummy, rows.at[b], sg).wait()
                pltpu.async_copy(rows.at[b], acc.at[didx.at[b]], ss, add=True)

        def drain_scatters(ss):
            for b in range(_K):
                pltpu.make_async_copy(dummy, rows0.at[b], ss).wait()

        bufs = ((sidx0, didx0, rows0, sg0, ss0, si0),
                (sidx1, didx1, rows1, sg1, ss1, si1))

        def phase(i, p):
            sidx, didx, rows, sg, ss, si = bufs[p]
            q = 1 - p
            sidxq, didxq, rowsq, sgq, ssq, siq = bufs[q]

            @pl.when((i >= 2) & (i <= G + 1))
            def _():
                drain_scatters(ss)

            @pl.when(i < G)
            def _():
                fire_idx(i, sidx, didx, si)

            @pl.when((i >= 1) & (i <= G))
            def _():
                drain_and_scatter(didxq, rowsq, sgq, ssq)

            @pl.when(i < G)
            def _():
                gather(sidx, rows, si, sg)

        def body(t, c):
            phase(2 * t, 0)
            phase(2 * t + 1, 1)
            return c
        lax.fori_loop(0, (G + 3) // 2, body, 0)

        plsc.subcore_barrier()
        _writeout(acc, o0_hbm, o1_hbm, cid, sid, N)

    return k


_ROWS = 5000


@functools.lru_cache(maxsize=None)
def _dense1(N):
    G = N // _ROWS

    def body(s0, s1, c0, c1, x, wla, wlb, b, wr, out, oinv):
        r = 1.0 / jnp.maximum(c0[:, :1] + c1[:, :1], 1.0)
        oinv[...] = r
        z = jnp.dot(s0[...] * r, wla[...], preferred_element_type=jnp.float32)
        z += jnp.dot(s1[...] * r, wlb[...], preferred_element_type=jnp.float32)
        z += jnp.dot(x[...], wr[...], preferred_element_type=jnp.float32)
        out[...] = jnp.maximum(z + b[...], 0.0)

    return pl.pallas_call(
        body,
        grid=(G,),
        in_specs=[pl.BlockSpec((_ROWS, _L), lambda i: (i, 0))] * 4
        + [pl.BlockSpec((_ROWS, 2 * _L), lambda i: (i, 0)),
           pl.BlockSpec((_L, 2 * _L), lambda i: (0, 0)),
           pl.BlockSpec((_L, 2 * _L), lambda i: (0, 0)),
           pl.BlockSpec((1, 2 * _L), lambda i: (0, 0)),
           pl.BlockSpec((2 * _L, 2 * _L), lambda i: (0, 0))],
        out_specs=[pl.BlockSpec((_ROWS, 2 * _L), lambda i: (i, 0)),
                   pl.BlockSpec((_ROWS, 1), lambda i: (i, 0))],
        out_shape=[jax.ShapeDtypeStruct((N, 2 * _L), jnp.float32),
                   jax.ShapeDtypeStruct((N, 1), jnp.float32)],
    )


@functools.lru_cache(maxsize=None)
def _dense2(N):
    G = N // _ROWS

    def body(s0, s1, inv, h, wla, wlb, b, wr, out):
        r = inv[...]
        z = jnp.dot(s0[...] * r, wla[...], preferred_element_type=jnp.float32)
        z += jnp.dot(s1[...] * r, wlb[...], preferred_element_type=jnp.float32)
        z += jnp.dot(h[...], wr[...], preferred_element_type=jnp.float32)
        out[...] = z + b[...]

    return pl.pallas_call(
        body,
        grid=(G,),
        in_specs=[pl.BlockSpec((_ROWS, _L), lambda i: (i, 0))] * 2
        + [pl.BlockSpec((_ROWS, 1), lambda i: (i, 0)),
           pl.BlockSpec((_ROWS, 2 * _L), lambda i: (i, 0)),
           pl.BlockSpec((_L, 2 * _L), lambda i: (0, 0)),
           pl.BlockSpec((_L, 2 * _L), lambda i: (0, 0)),
           pl.BlockSpec((1, 2 * _L), lambda i: (0, 0)),
           pl.BlockSpec((2 * _L, 2 * _L), lambda i: (0, 0))],
        out_specs=pl.BlockSpec((_ROWS, 2 * _L), lambda i: (i, 0)),
        out_shape=jax.ShapeDtypeStruct((N, 2 * _L), jnp.float32),
    )


def kernel(x, edge_index, W_l1, b_l1, W_r1, W_l2, b_l2, W_r2, batch_size):
    N, D = x.shape
    E = edge_index.shape[1]
    src = edge_index[0]
    dst = edge_index[1]

    xr = x.reshape(2 * N, _L)
    Epad = _pad_chunks(E) * _CHUNK
    src_pad = jnp.concatenate(
        [src, jnp.zeros((Epad - E,), jnp.int32)])
    dst2d = jnp.concatenate(
        [dst, jnp.full((Epad - E,), N, jnp.int32)]).reshape(-1, _CHUNK)

    c0, c1 = _count_kernel(N, E)(dst2d)
    s10, s11 = _gather_scatter_kernel(N, E)(src_pad, dst2d, xr)
    h, inv = _dense1(N)(
        s10, s11, c0, c1, x,
        W_l1[:_L, :], W_l1[_L:, :], b_l1.reshape(1, 2 * _L), W_r1)
    s20, s21 = _gather_scatter_kernel(N, E)(src_pad, dst2d, h.reshape(2 * N, _L))
    out = _dense2(N)(
        s20, s21, inv, h,
        W_l2[:_L, :], W_l2[_L:, :], b_l2.reshape(1, 2 * _L), W_r2)
    return out


# speedup vs baseline: 1.0289x; 1.0289x over previous
import functools

import jax
import jax.numpy as jnp
from jax import lax
from jax.experimental import pallas as pl
from jax.experimental.pallas import tpu as pltpu
from jax.experimental.pallas import tpu_sc as plsc

_NC = 2
_NS = 16
_L = 16
_CHUNK = 128
_ZR = 256


def _row_partition(N):
    per = (N // _NS) & ~7
    extra = N - _NS * per
    return per, extra


def _zero_accumulator(zbuf, acc, sid, N):
    per, extra = _row_partition(N)
    def initz(i, c):
        zbuf[i, :] = jnp.zeros((_L,), jnp.float32)
        return c
    lax.fori_loop(0, _ZR, initz, 0)
    base = sid * per
    span = per + extra
    nz, zt = divmod(span, _ZR)
    def zc(j, c):
        pltpu.sync_copy(zbuf, acc.at[pl.ds(base + j * _ZR, _ZR)])
        return c
    lax.fori_loop(0, nz, zc, 0)
    if zt:
        pltpu.sync_copy(zbuf.at[pl.ds(0, zt)], acc.at[pl.ds(base + nz * _ZR, zt)])


def _writeout(acc, o0, o1, cid, sid, N):
    per, extra = _row_partition(N)
    base = sid * per
    sl = pl.ds(base, per)
    tl = pl.ds(_NS * per, max(extra, 8))

    @pl.when(cid == 0)
    def _():
        pltpu.sync_copy(acc.at[sl], o0.at[sl])

    @pl.when(cid == 1)
    def _():
        pltpu.sync_copy(acc.at[sl], o1.at[sl])

    if extra:
        @pl.when((cid == 0) & (sid == _NS - 1))
        def _():
            pltpu.sync_copy(acc.at[tl], o0.at[tl])

        @pl.when((cid == 1) & (sid == _NS - 1))
        def _():
            pltpu.sync_copy(acc.at[tl], o1.at[tl])


_K = 4


def _pad_chunks(E):
    q = _NC * _NS * _K
    return -(-(E // _CHUNK + (1 if E % _CHUNK else 0)) // q) * q


@functools.lru_cache(maxsize=None)
def _count_kernel(N, E):
    C = _pad_chunks(E)
    G = C // (_NC * _NS * _K)
    mesh = plsc.VectorSubcoreMesh(core_axis_name="c", subcore_axis_name="s")

    @functools.partial(
        pl.kernel, mesh=mesh,
        out_type=(jax.ShapeDtypeStruct((N, _L), jnp.float32),
                  jax.ShapeDtypeStruct((N, _L), jnp.float32)),
        scratch_types=[
            pltpu.VMEM((_K, _CHUNK), jnp.int32),
            pltpu.VMEM((_K, _CHUNK), jnp.int32),
            pltpu.VMEM((_CHUNK, _L), jnp.float32),
            pltpu.VMEM((_ZR, _L), jnp.float32),
            pltpu.VMEM_SHARED((N + 8, _L), jnp.float32),
            pltpu.SemaphoreType.DMA,
            pltpu.SemaphoreType.DMA,
        ],
        compiler_params=pltpu.CompilerParams(use_tc_tiling_on_sc=False),
    )
    def k(dst2d_hbm, c0_hbm, c1_hbm, didx0, didx1, ones, zbuf, acc, ss0, ss1):
        cid = lax.axis_index("c")
        sid = lax.axis_index("s")

        def inito(i, c):
            ones[i, :] = jnp.full((_L,), 1.0, jnp.float32)
            return c
        lax.fori_loop(0, _CHUNK, inito, 0)
        _zero_accumulator(zbuf, acc, sid, N)
        plsc.subcore_barrier()

        row0 = cid * (C // _NC) + sid * (G * _K)
        dummy = c0_hbm.at[pl.ds(0, _CHUNK)]

        def phase(i, didx, ss):
            @pl.when((i >= 2) & (i <= G + 1))
            def _():
                for b in range(_K):
                    pltpu.make_async_copy(dummy, ones, ss).wait()

            @pl.when(i < G)
            def _():
                pltpu.sync_copy(dst2d_hbm.at[pl.ds(row0 + i * _K, _K)], didx)
                for b in range(_K):
                    pltpu.async_copy(ones, acc.at[didx.at[b]], ss, add=True)

        def body(t, c):
            phase(2 * t, didx0, ss0)
            phase(2 * t + 1, didx1, ss1)
            return c
        lax.fori_loop(0, (G + 3) // 2, body, 0)

        plsc.subcore_barrier()
        _writeout(acc, c0_hbm, c1_hbm, cid, sid, N)

    return k


@functools.lru_cache(maxsize=None)
def _gather_scatter_kernel(N, E):
    C = _pad_chunks(E)
    G = C // (_NS * _K)
    mesh = plsc.VectorSubcoreMesh(core_axis_name="c", subcore_axis_name="s")

    @functools.partial(
        pl.kernel, mesh=mesh,
        out_type=(jax.ShapeDtypeStruct((N, _L), jnp.float32),
                  jax.ShapeDtypeStruct((N, _L), jnp.float32)),
        scratch_types=[
            pltpu.VMEM((_K * _CHUNK,), jnp.int32),
            pltpu.VMEM((_K * _CHUNK,), jnp.int32),
            pltpu.VMEM((_K, _CHUNK), jnp.int32),
            pltpu.VMEM((_K, _CHUNK), jnp.int32),
            pltpu.VMEM((_K, _CHUNK, _L), jnp.float32),
            pltpu.VMEM((_K, _CHUNK, _L), jnp.float32),
            pltpu.VMEM((_ZR, _L), jnp.float32),
            pltpu.VMEM_SHARED((N + 8, _L), jnp.float32),
            pltpu.SemaphoreType.DMA,
            pltpu.SemaphoreType.DMA,
            pltpu.SemaphoreType.DMA,
            pltpu.SemaphoreType.DMA,
            pltpu.SemaphoreType.DMA,
            pltpu.SemaphoreType.DMA,
        ],
        compiler_params=pltpu.CompilerParams(use_tc_tiling_on_sc=False),
    )
    def k(src_hbm, dst2d_hbm, tab_hbm, o0_hbm, o1_hbm,
          sidx0, sidx1, didx0, didx1, rows0, rows1, zbuf, acc,
          sg0, sg1, ss0, ss1, si0, si1):
        cid = lax.axis_index("c")
        sid = lax.axis_index("s")

        _zero_accumulator(zbuf, acc, sid, N)
        plsc.subcore_barrier()

        chunk0 = sid * (G * _K)
        off = cid
        dummy = o0_hbm.at[pl.ds(0, _CHUNK)]
        dummy_s = src_hbm.at[pl.ds(0, _K * _CHUNK)]
        dummy_d = dst2d_hbm.at[pl.ds(0, _K)]

        def fire_idx(i, sidx, didx, si):
            e0 = pl.multiple_of(chunk0 * _CHUNK + i * (_K * _CHUNK), 8)
            pltpu.async_copy(src_hbm.at[pl.ds(e0, _K * _CHUNK)], sidx, si)
            pltpu.async_copy(dst2d_hbm.at[pl.ds(chunk0 + i * _K, _K)],
                             didx, si)

        def gather(sidx, rows, si, sg):
            pltpu.make_async_copy(dummy_s, sidx, si).wait()
            pltpu.make_async_copy(dummy_d, didx0, si).wait()

            def f(j, c):
                sl = pl.ds(j * _L, _L)
                sidx[sl] = sidx[sl] * 2 + off
                return c
            lax.fori_loop(0, _K * _CHUNK // _L, f, 0)
            for b in range(_K):
                pltpu.async_copy(
                    tab_hbm.at[sidx.at[pl.ds(b * _CHUNK, _CHUNK)]],
                    rows.at[b], sg)

        def drain_and_scatter(didx, rows, sg, ss):
            for b in range(_K):
                pltpu.make_async_copy(dummy, rows.at[b], sg).wait()
                pltpu.async_copy(rows.at[b], acc.at[didx.at[b]], ss, add=True)

        def drain_scatters(ss):
            for b in range(_K):
                pltpu.make_async_copy(dummy, rows0.at[b], ss).wait()

        bufs = ((sidx0, didx0, rows0, sg0, ss0, si0),
                (sidx1, didx1, rows1, sg1, ss1, si1))

        def phase(i, p):
            sidx, didx, rows, sg, ss, si = bufs[p]
            q = 1 - p
            sidxq, didxq, rowsq, sgq, ssq, siq = bufs[q]

            @pl.when((i >= 2) & (i <= G + 1))
            def _():
                drain_scatters(ss)

            @pl.when(i < G)
            def _():
                fire_idx(i, sidx, didx, si)

            @pl.when((i >= 1) & (i <= G))
            def _():
                drain_and_scatter(didxq, rowsq, sgq, ssq)

            @pl.when(i < G)
            def _():
                gather(sidx, rows, si, sg)

        def body(t, c):
            phase(2 * t, 0)
            phase(2 * t + 1, 1)
            return c
        lax.fori_loop(0, (G + 3) // 2, body, 0)

        plsc.subcore_barrier()
        _writeout(acc, o0_hbm, o1_hbm, cid, sid, N)

    return k


_ROWS = 10000
_IROWS = 10000


@functools.lru_cache(maxsize=None)
def _inv_kernel(N):
    G = N // _IROWS

    def body(c0, c1, out):
        out[...] = 1.0 / jnp.maximum(c0[:, :1] + c1[:, :1], 1.0)

    return pl.pallas_call(
        body,
        grid=(G,),
        in_specs=[pl.BlockSpec((_IROWS, _L), lambda i: (i, 0))] * 2,
        out_specs=pl.BlockSpec((_IROWS, 1), lambda i: (i, 0)),
        out_shape=jax.ShapeDtypeStruct((N, 1), jnp.float32),
    )


@functools.lru_cache(maxsize=None)
def _dense1(N):
    G = N // _ROWS

    def body(s0, s1, inv, x, wla, wlb, b, wr, out):
        r = inv[...]
        z = jnp.dot(s0[...] * r, wla[...], preferred_element_type=jnp.float32)
        z += jnp.dot(s1[...] * r, wlb[...], preferred_element_type=jnp.float32)
        z += jnp.dot(x[...], wr[...], preferred_element_type=jnp.float32)
        out[...] = jnp.maximum(z + b[...], 0.0)

    return pl.pallas_call(
        body,
        grid=(G,),
        in_specs=[pl.BlockSpec((_ROWS, _L), lambda i: (i, 0))] * 2
        + [pl.BlockSpec((_ROWS, 1), lambda i: (i, 0)),
           pl.BlockSpec((_ROWS, 2 * _L), lambda i: (i, 0)),
           pl.BlockSpec((_L, 2 * _L), lambda i: (0, 0)),
           pl.BlockSpec((_L, 2 * _L), lambda i: (0, 0)),
           pl.BlockSpec((1, 2 * _L), lambda i: (0, 0)),
           pl.BlockSpec((2 * _L, 2 * _L), lambda i: (0, 0))],
        out_specs=pl.BlockSpec((_ROWS, 2 * _L), lambda i: (i, 0)),
        out_shape=jax.ShapeDtypeStruct((N, 2 * _L), jnp.float32),
    )


@functools.lru_cache(maxsize=None)
def _dense2(N):
    G = N // _ROWS

    def body(s0, s1, inv, h, wla, wlb, b, wr, out):
        r = inv[...]
        z = jnp.dot(s0[...] * r, wla[...], preferred_element_type=jnp.float32)
        z += jnp.dot(s1[...] * r, wlb[...], preferred_element_type=jnp.float32)
        z += jnp.dot(h[...], wr[...], preferred_element_type=jnp.float32)
        out[...] = z + b[...]

    return pl.pallas_call(
        body,
        grid=(G,),
        in_specs=[pl.BlockSpec((_ROWS, _L), lambda i: (i, 0))] * 2
        + [pl.BlockSpec((_ROWS, 1), lambda i: (i, 0)),
           pl.BlockSpec((_ROWS, 2 * _L), lambda i: (i, 0)),
           pl.BlockSpec((_L, 2 * _L), lambda i: (0, 0)),
           pl.BlockSpec((_L, 2 * _L), lambda i: (0, 0)),
           pl.BlockSpec((1, 2 * _L), lambda i: (0, 0)),
           pl.BlockSpec((2 * _L, 2 * _L), lambda i: (0, 0))],
        out_specs=pl.BlockSpec((_ROWS, 2 * _L), lambda i: (i, 0)),
        out_shape=jax.ShapeDtypeStruct((N, 2 * _L), jnp.float32),
    )


def kernel(x, edge_index, W_l1, b_l1, W_r1, W_l2, b_l2, W_r2, batch_size):
    N, D = x.shape
    E = edge_index.shape[1]
    src = edge_index[0]
    dst = edge_index[1]

    xr = x.reshape(2 * N, _L)
    Epad = _pad_chunks(E) * _CHUNK
    src_pad = jnp.concatenate(
        [src, jnp.zeros((Epad - E,), jnp.int32)])
    dst2d = jnp.concatenate(
        [dst, jnp.full((Epad - E,), N, jnp.int32)]).reshape(-1, _CHUNK)

    c0, c1 = _count_kernel(N, E)(dst2d)
    inv = _inv_kernel(N)(c0, c1)
    s10, s11 = _gather_scatter_kernel(N, E)(src_pad, dst2d, xr)
    h = _dense1(N)(
        s10, s11, inv, x,
        W_l1[:_L, :], W_l1[_L:, :], b_l1.reshape(1, 2 * _L), W_r1)
    s20, s21 = _gather_scatter_kernel(N, E)(src_pad, dst2d, h.reshape(2 * N, _L))
    out = _dense2(N)(
        s20, s21, inv, h,
        W_l2[:_L, :], W_l2[_L:, :], b_l2.reshape(1, 2 * _L), W_r2)
    return out


# speedup vs baseline: 1.0515x; 1.0219x over previous
import functools

import jax
import jax.numpy as jnp
from jax import lax
from jax.experimental import pallas as pl
from jax.experimental.pallas import tpu as pltpu
from jax.experimental.pallas import tpu_sc as plsc

_NC = 2
_NS = 16
_L = 16
_CHUNK = 128
_ZR = 256


def _row_partition(N):
    per = (N // _NS) & ~7
    extra = N - _NS * per
    return per, extra


def _zero_accumulator(zbuf, acc, sid, N):
    per, extra = _row_partition(N)
    def initz(i, c):
        zbuf[i, :] = jnp.zeros((_L,), jnp.float32)
        return c
    lax.fori_loop(0, _ZR, initz, 0)
    base = sid * per
    span = per + extra
    nz, zt = divmod(span, _ZR)
    def zc(j, c):
        pltpu.sync_copy(zbuf, acc.at[pl.ds(base + j * _ZR, _ZR)])
        return c
    lax.fori_loop(0, nz, zc, 0)
    if zt:
        pltpu.sync_copy(zbuf.at[pl.ds(0, zt)], acc.at[pl.ds(base + nz * _ZR, zt)])


def _writeout(acc, o0, o1, cid, sid, N):
    per, extra = _row_partition(N)
    base = sid * per
    sl = pl.ds(base, per)
    tl = pl.ds(_NS * per, max(extra, 8))

    @pl.when(cid == 0)
    def _():
        pltpu.sync_copy(acc.at[sl], o0.at[sl])

    @pl.when(cid == 1)
    def _():
        pltpu.sync_copy(acc.at[sl], o1.at[sl])

    if extra:
        @pl.when((cid == 0) & (sid == _NS - 1))
        def _():
            pltpu.sync_copy(acc.at[tl], o0.at[tl])

        @pl.when((cid == 1) & (sid == _NS - 1))
        def _():
            pltpu.sync_copy(acc.at[tl], o1.at[tl])


_K = 4


def _pad_chunks(E):
    q = _NC * _NS * _K
    return -(-(E // _CHUNK + (1 if E % _CHUNK else 0)) // q) * q


@functools.lru_cache(maxsize=None)
def _count_kernel(N, E):
    C = _pad_chunks(E)
    G = C // (_NC * _NS * _K)
    mesh = plsc.VectorSubcoreMesh(core_axis_name="c", subcore_axis_name="s")

    @functools.partial(
        pl.kernel, mesh=mesh,
        out_type=(jax.ShapeDtypeStruct((N, _L), jnp.float32),
                  jax.ShapeDtypeStruct((N, _L), jnp.float32)),
        scratch_types=[
            pltpu.VMEM((_K, _CHUNK), jnp.int32),
            pltpu.VMEM((_K, _CHUNK), jnp.int32),
            pltpu.VMEM((_CHUNK, _L), jnp.float32),
            pltpu.VMEM((_ZR, _L), jnp.float32),
            pltpu.VMEM_SHARED((N + 8, _L), jnp.float32),
            pltpu.SemaphoreType.DMA,
            pltpu.SemaphoreType.DMA,
        ],
        compiler_params=pltpu.CompilerParams(use_tc_tiling_on_sc=False),
    )
    def k(dst2d_hbm, c0_hbm, c1_hbm, didx0, didx1, ones, zbuf, acc, ss0, ss1):
        cid = lax.axis_index("c")
        sid = lax.axis_index("s")

        def inito(i, c):
            ones[i, :] = jnp.full((_L,), 1.0, jnp.float32)
            return c
        lax.fori_loop(0, _CHUNK, inito, 0)
        _zero_accumulator(zbuf, acc, sid, N)
        plsc.subcore_barrier()

        row0 = cid * (C // _NC) + sid * (G * _K)
        dummy = c0_hbm.at[pl.ds(0, _CHUNK)]

        def phase(i, didx, ss):
            @pl.when((i >= 2) & (i <= G + 1))
            def _():
                for b in range(_K):
                    pltpu.make_async_copy(dummy, ones, ss).wait()

            @pl.when(i < G)
            def _():
                pltpu.sync_copy(dst2d_hbm.at[pl.ds(row0 + i * _K, _K)], didx)
                for b in range(_K):
                    pltpu.async_copy(ones, acc.at[didx.at[b]], ss, add=True)

        def body(t, c):
            phase(2 * t, didx0, ss0)
            phase(2 * t + 1, didx1, ss1)
            return c
        lax.fori_loop(0, (G + 3) // 2, body, 0)

        plsc.subcore_barrier()
        _writeout(acc, c0_hbm, c1_hbm, cid, sid, N)

    return k


@functools.lru_cache(maxsize=None)
def _gather_scatter_kernel(N, E):
    C = _pad_chunks(E)
    G = C // (_NS * _K)
    mesh = plsc.VectorSubcoreMesh(core_axis_name="c", subcore_axis_name="s")

    @functools.partial(
        pl.kernel, mesh=mesh,
        out_type=jax.ShapeDtypeStruct((N, 2 * _L), jnp.float32),
        scratch_types=[
            pltpu.VMEM((_K * _CHUNK,), jnp.int32),
            pltpu.VMEM((_K * _CHUNK,), jnp.int32),
            pltpu.VMEM((_K, _CHUNK), jnp.int32),
            pltpu.VMEM((_K, _CHUNK), jnp.int32),
            pltpu.VMEM((_K, _CHUNK, _L), jnp.float32),
            pltpu.VMEM((_K, _CHUNK, _L), jnp.float32),
            pltpu.VMEM((_ZR, _L), jnp.float32),
            pltpu.VMEM_SHARED((N + 8, _L), jnp.float32),
            pltpu.SemaphoreType.DMA,
            pltpu.SemaphoreType.DMA,
            pltpu.SemaphoreType.DMA,
            pltpu.SemaphoreType.DMA,
            pltpu.SemaphoreType.DMA,
            pltpu.SemaphoreType.DMA,
        ],
        compiler_params=pltpu.CompilerParams(use_tc_tiling_on_sc=False),
    )
    def k(src_hbm, dst2d_hbm, tab_hbm, o_hbm,
          sidx0, sidx1, didx0, didx1, rows0, rows1, zbuf, acc,
          sg0, sg1, ss0, ss1, si0, si1):
        cid = lax.axis_index("c")
        sid = lax.axis_index("s")

        _zero_accumulator(zbuf, acc, sid, N)
        plsc.subcore_barrier()

        chunk0 = sid * (G * _K)
        off = cid
        dummy = o_hbm.at[pl.ds(0, _CHUNK), pl.ds(0, _L)]
        dummy_s = src_hbm.at[pl.ds(0, _K * _CHUNK)]
        dummy_d = dst2d_hbm.at[pl.ds(0, _K)]

        def fire_idx(i, sidx, didx, si):
            e0 = pl.multiple_of(chunk0 * _CHUNK + i * (_K * _CHUNK), 8)
            pltpu.async_copy(src_hbm.at[pl.ds(e0, _K * _CHUNK)], sidx, si)
            pltpu.async_copy(dst2d_hbm.at[pl.ds(chunk0 + i * _K, _K)],
                             didx, si)

        def gather(sidx, rows, si, sg):
            pltpu.make_async_copy(dummy_s, sidx, si).wait()
            pltpu.make_async_copy(dummy_d, didx0, si).wait()

            def f(j, c):
                sl = pl.ds(j * _L, _L)
                sidx[sl] = sidx[sl] * 2 + off
                return c
            lax.fori_loop(0, _K * _CHUNK // _L, f, 0)
            for b in range(_K):
                pltpu.async_copy(
                    tab_hbm.at[sidx.at[pl.ds(b * _CHUNK, _CHUNK)]],
                    rows.at[b], sg)

        def drain_and_scatter(didx, rows, sg, ss):
            for b in range(_K):
                pltpu.make_async_copy(dummy, rows.at[b], sg).wait()
                pltpu.async_copy(rows.at[b], acc.at[didx.at[b]], ss, add=True)

        def drain_scatters(ss):
            for b in range(_K):
                pltpu.make_async_copy(dummy, rows0.at[b], ss).wait()

        bufs = ((sidx0, didx0, rows0, sg0, ss0, si0),
                (sidx1, didx1, rows1, sg1, ss1, si1))

        def phase(i, p):
            sidx, didx, rows, sg, ss, si = bufs[p]
            q = 1 - p
            sidxq, didxq, rowsq, sgq, ssq, siq = bufs[q]

            @pl.when((i >= 2) & (i <= G + 1))
            def _():
                drain_scatters(ss)

            @pl.when(i < G)
            def _():
                fire_idx(i, sidx, didx, si)

            @pl.when((i >= 1) & (i <= G))
            def _():
                drain_and_scatter(didxq, rowsq, sgq, ssq)

            @pl.when(i < G)
            def _():
                gather(sidx, rows, si, sg)

        def body(t, c):
            phase(2 * t, 0)
            phase(2 * t + 1, 1)
            return c
        lax.fori_loop(0, (G + 3) // 2, body, 0)

        plsc.subcore_barrier()
        per, extra = _row_partition(N)
        base = sid * per
        cs = pl.ds(off * _L, _L)
        pltpu.sync_copy(acc.at[pl.ds(base, per)],
                        o_hbm.at[pl.ds(base, per), cs])
        if extra:
            @pl.when(sid == _NS - 1)
            def _():
                pltpu.sync_copy(acc.at[pl.ds(_NS * per, extra)],
                                o_hbm.at[pl.ds(_NS * per, extra), cs])

    return k


_ROWS = 10000
_IROWS = 10000


@functools.lru_cache(maxsize=None)
def _inv_kernel(N):
    G = N // _IROWS

    def body(c0, c1, out):
        out[...] = 1.0 / jnp.maximum(c0[:, :1] + c1[:, :1], 1.0)

    return pl.pallas_call(
        body,
        grid=(G,),
        in_specs=[pl.BlockSpec((_IROWS, _L), lambda i: (i, 0))] * 2,
        out_specs=pl.BlockSpec((_IROWS, 1), lambda i: (i, 0)),
        out_shape=jax.ShapeDtypeStruct((N, 1), jnp.float32),
    )


@functools.lru_cache(maxsize=None)
def _dense1(N):
    G = N // _ROWS

    def body(s, inv, x, wl, b, wr, out):
        r = inv[...]
        z = jnp.dot(s[...] * r, wl[...], preferred_element_type=jnp.float32)
        z += jnp.dot(x[...], wr[...], preferred_element_type=jnp.float32)
        out[...] = jnp.maximum(z + b[...], 0.0)

    return pl.pallas_call(
        body,
        grid=(G,),
        in_specs=[pl.BlockSpec((_ROWS, 2 * _L), lambda i: (i, 0)),
                  pl.BlockSpec((_ROWS, 1), lambda i: (i, 0)),
                  pl.BlockSpec((_ROWS, 2 * _L), lambda i: (i, 0)),
                  pl.BlockSpec((2 * _L, 2 * _L), lambda i: (0, 0)),
                  pl.BlockSpec((1, 2 * _L), lambda i: (0, 0)),
                  pl.BlockSpec((2 * _L, 2 * _L), lambda i: (0, 0))],
        out_specs=pl.BlockSpec((_ROWS, 2 * _L), lambda i: (i, 0)),
        out_shape=jax.ShapeDtypeStruct((N, 2 * _L), jnp.float32),
    )


@functools.lru_cache(maxsize=None)
def _dense2(N):
    G = N // _ROWS

    def body(s, inv, h, wl, b, wr, out):
        r = inv[...]
        z = jnp.dot(s[...] * r, wl[...], preferred_element_type=jnp.float32)
        z += jnp.dot(h[...], wr[...], preferred_element_type=jnp.float32)
        out[...] = z + b[...]

    return pl.pallas_call(
        body,
        grid=(G,),
        in_specs=[pl.BlockSpec((_ROWS, 2 * _L), lambda i: (i, 0)),
                  pl.BlockSpec((_ROWS, 1), lambda i: (i, 0)),
                  pl.BlockSpec((_ROWS, 2 * _L), lambda i: (i, 0)),
                  pl.BlockSpec((2 * _L, 2 * _L), lambda i: (0, 0)),
                  pl.BlockSpec((1, 2 * _L), lambda i: (0, 0)),
                  pl.BlockSpec((2 * _L, 2 * _L), lambda i: (0, 0))],
        out_specs=pl.BlockSpec((_ROWS, 2 * _L), lambda i: (i, 0)),
        out_shape=jax.ShapeDtypeStruct((N, 2 * _L), jnp.float32),
    )


def kernel(x, edge_index, W_l1, b_l1, W_r1, W_l2, b_l2, W_r2, batch_size):
    N, D = x.shape
    E = edge_index.shape[1]
    src = edge_index[0]
    dst = edge_index[1]

    xr = x.reshape(2 * N, _L)
    Epad = _pad_chunks(E) * _CHUNK
    src_pad = jnp.concatenate(
        [src, jnp.zeros((Epad - E,), jnp.int32)])
    dst2d = jnp.concatenate(
        [dst, jnp.full((Epad - E,), N, jnp.int32)]).reshape(-1, _CHUNK)

    c0, c1 = _count_kernel(N, E)(dst2d)
    inv = _inv_kernel(N)(c0, c1)
    s1 = _gather_scatter_kernel(N, E)(src_pad, dst2d, xr)
    h = _dense1(N)(s1, inv, x, W_l1, b_l1.reshape(1, 2 * _L), W_r1)
    s2 = _gather_scatter_kernel(N, E)(src_pad, dst2d, h.reshape(2 * N, _L))
    out = _dense2(N)(s2, inv, h, W_l2, b_l2.reshape(1, 2 * _L), W_r2)
    return out
